# Initial kernel scaffold; baseline (speedup 1.0000x reference)
import jax
import jax.numpy as jnp
from jax.experimental import pallas as pl


def kernel(boxes, scores):
    raise NotImplementedError("write your pallas kernel here")


# speedup vs baseline: 19.6211x; 19.6211x over previous
import functools

import jax
import jax.numpy as jnp
from jax import lax
from jax.experimental import pallas as pl
from jax.experimental.pallas import tpu as pltpu

_NMS_T = 0.5
_SCORE_T = 0.05
_B = 128


def _iou_hot(sx1, sy1, sx2, sy2, sa, tx1, ty1, tx2, ty2, ta):
    xx1 = jnp.maximum(sx1, tx1)
    yy1 = jnp.maximum(sy1, ty1)
    xx2 = jnp.minimum(sx2, tx2)
    yy2 = jnp.minimum(sy2, ty2)
    inter = jnp.maximum(xx2 - xx1, 0.0) * jnp.maximum(yy2 - yy1, 0.0)
    union = sa + ta - inter
    iou = inter / (union + 1e-6)
    return (iou > _NMS_T).astype(jnp.float32)


def _area(x1, y1, x2, y2):
    return jnp.maximum(x2 - x1, 0.0) * jnp.maximum(y2 - y1, 0.0)


def _nms_body(x1r, y1r, x2r, y2r, sr, x1c, y1c, x2c, y2c, keep, e_scr):
    nb = x1r.shape[0]
    keep[...] = (sr[...] > _SCORE_T).astype(jnp.float32)

    lane = lax.broadcasted_iota(jnp.int32, (_B, _B), 1)
    sub = lax.broadcasted_iota(jnp.int32, (_B, _B), 0)
    triu = (lane > sub).astype(jnp.float32)
    eye = (lane == sub).astype(jnp.float32)
    lane_row = lax.broadcasted_iota(jnp.int32, (1, _B), 1)

    def outer(bi, carry):
        sx1 = x1c[bi]
        sy1 = y1c[bi]
        sx2 = x2c[bi]
        sy2 = y2c[bi]
        sa = _area(sx1, sy1, sx2, sy2)
        tx1 = x1r[bi]
        ty1 = y1r[bi]
        tx2 = x2r[bi]
        ty2 = y2r[bi]
        ta = _area(tx1, ty1, tx2, ty2)
        e_scr[...] = _iou_hot(sx1, sy1, sx2, sy2, sa, tx1, ty1, tx2, ty2, ta) * triu

        k0 = keep[bi]

        def inner(r, k):
            row = e_scr[pl.ds(r, 1), :]
            onehot = (lane_row == r).astype(jnp.float32)
            krb = jnp.max(k * onehot, axis=1, keepdims=True)
            return k * (1.0 - row * krb)

        k = lax.fori_loop(0, _B, inner, k0)
        keep[bi] = k
        kcol = jnp.sum(jnp.broadcast_to(k, (_B, _B)) * eye, axis=1, keepdims=True)

        def inner2(bj, c2):
            ux1 = x1r[bj]
            uy1 = y1r[bj]
            ux2 = x2r[bj]
            uy2 = y2r[bj]
            ua = _area(ux1, uy1, ux2, uy2)
            hot = _iou_hot(sx1, sy1, sx2, sy2, sa, ux1, uy1, ux2, uy2, ua)
            sup = jnp.max(hot * kcol, axis=0, keepdims=True)
            keep[bj] = keep[bj] * (1.0 - sup)
            return c2

        lax.fori_loop(bi + 1, nb, inner2, 0)
        return carry

    lax.fori_loop(0, nb, outer, 0)


def _blocked_nms(x1r, y1r, x2r, y2r, sr, x1c, y1c, x2c, y2c):
    nb = x1r.shape[0]
    return pl.pallas_call(
        _nms_body,
        out_shape=jax.ShapeDtypeStruct((nb, 1, _B), jnp.float32),
        scratch_shapes=[pltpu.VMEM((_B, _B), jnp.float32)],
    )(x1r, y1r, x2r, y2r, sr, x1c, y1c, x2c, y2c)


def kernel(boxes, scores):
    n = scores.shape[0]
    nb = (n + _B - 1) // _B
    np_ = nb * _B

    order = jnp.argsort(-scores)
    b = boxes[order]
    s = scores[order]

    bp = jnp.pad(b, ((0, np_ - n), (0, 0)))
    sp = jnp.pad(s, ((0, np_ - n),), constant_values=-1.0)

    x1, y1, x2, y2 = bp[:, 0], bp[:, 1], bp[:, 2], bp[:, 3]

    def rform(v):
        return v.reshape(nb, 1, _B)

    def cform(v):
        return v.reshape(nb, _B, 1)

    keep_f = _blocked_nms(
        rform(x1), rform(y1), rform(x2), rform(y2), rform(sp),
        cform(x1), cform(y1), cform(x2), cform(y2),
    )
    keep_sorted = keep_f.reshape(np_)[:n] > 0.5
    kept_scores_sorted = s * keep_sorted.astype(s.dtype)

    idx = jnp.arange(n, dtype=jnp.int32)
    inv = jnp.zeros(n, dtype=jnp.int32).at[order].set(idx)
    kept_scores = kept_scores_sorted[inv]
    keep_orig = keep_sorted[inv]
    return kept_scores, keep_orig
